# baseline (speedup 1.0000x reference)
import functools

import ml_dtypes
import numpy as np

import jax
import jax.numpy as jnp
from jax import lax
from jax.experimental import pallas as pl
from jax.experimental.pallas import tpu as pltpu
from jax.experimental.pallas import tpu_sc as plsc

_L = 16
_NC = 2
_NS = 16
_NW = _NC * _NS
_N = 8 * 3 * 224 * 224
_PER_W = _N // _NW
_NCH = 8
_CH = _PER_W // _NCH
_CVECS = _CH // _L


def _decision_thresholds():
    lev = np.array([0.0, 0.2, 0.4, 0.6, 0.8, 1.0], np.float64)
    t32 = lev.astype(np.float32)
    bt = t32.astype(ml_dtypes.bfloat16).astype(np.float64)
    t2 = (t32 * t32).astype(np.float32).astype(np.float64)
    B = (t2[1:] - t2[:-1]) / (2.0 * (bt[1:] - bt[:-1]))
    C = []
    for b in B:
        vb = np.float64(ml_dtypes.bfloat16(b))
        bits = np.float32(vb).view(np.uint32) >> 16
        if vb > b:
            bits -= 1
            vb = np.float64(np.array([bits << 16], np.uint32).view(np.float32)[0])
        nxt = np.float64(np.array([(bits + 1) << 16], np.uint32).view(np.float32)[0])
        mid = np.float32((vb + nxt) / 2.0)
        if bits & 1:
            C.append(float(np.nextafter(mid, np.float32(-1.0), dtype=np.float32)))
        else:
            C.append(float(mid))
    return C


_C = _decision_thresholds()


def _sc_quantize(z_flat, levels):
    mesh = plsc.VectorSubcoreMesh(core_axis_name="c", subcore_axis_name="s")

    @functools.partial(
        pl.kernel,
        mesh=mesh,
        out_type=[
            jax.ShapeDtypeStruct((_N,), jnp.float32),
            jax.ShapeDtypeStruct((_NW * _L,), jnp.float32),
        ],
        scratch_types=[
            pltpu.VMEM((_CH,), jnp.float32),
            pltpu.VMEM((_CH,), jnp.float32),
            pltpu.VMEM((_CH,), jnp.float32),
            pltpu.VMEM((_CH,), jnp.float32),
            pltpu.VMEM((_L,), jnp.float32),
            pltpu.VMEM((_L,), jnp.float32),
            pltpu.SemaphoreType.DMA,
            pltpu.SemaphoreType.DMA,
            pltpu.SemaphoreType.DMA,
            pltpu.SemaphoreType.DMA,
        ],
    )
    def body(z_hbm, lvl_hbm, q_hbm, part_hbm,
             in0, in1, out0, out1, lvlbuf, pbuf,
             semi0, semi1, semo0, semo1):
        wid = lax.axis_index("c") * _NS + lax.axis_index("s")
        base = wid * _PER_W
        pltpu.sync_copy(lvl_hbm, lvlbuf)
        lvl_vec = lvlbuf[...]

        inb = [in0, in1]
        outb = [out0, out1]
        semi = [semi0, semi1]
        semo = [semo0, semo1]
        cp_in = [None, None]
        cp_out = [None, None]

        def chunk(xref, qref, acc):
            def step(j, acc):
                o = j * _L
                xv = xref[pl.ds(o, _L)]
                iv = (jnp.where(xv > _C[0], 1, 0)
                      + jnp.where(xv > _C[1], 1, 0)
                      + jnp.where(xv > _C[2], 1, 0)
                      + jnp.where(xv > _C[3], 1, 0)
                      + jnp.where(xv > _C[4], 1, 0))
                qv = lax.gather(
                    lvl_vec, iv[:, None],
                    dimension_numbers=lax.GatherDimensionNumbers(
                        offset_dims=(), collapsed_slice_dims=(0,),
                        start_index_map=(0,)),
                    slice_sizes=(1,),
                    mode=lax.GatherScatterMode.PROMISE_IN_BOUNDS)
                qref[pl.ds(o, _L)] = qv
                d = qv - xv
                return acc + d * d
            return lax.fori_loop(0, _CVECS, step, acc, unroll=4)

        cp_in[0] = pltpu.async_copy(z_hbm.at[pl.ds(base, _CH)], in0, semi0)
        acc = jnp.zeros((_L,), jnp.float32)
        for k in range(_NCH):
            cur = k & 1
            if k + 1 < _NCH:
                cp_in[1 - cur] = pltpu.async_copy(
                    z_hbm.at[pl.ds(base + (k + 1) * _CH, _CH)],
                    inb[1 - cur], semi[1 - cur])
            cp_in[cur].wait()
            if k >= 2:
                cp_out[cur].wait()
            acc = chunk(inb[cur], outb[cur], acc)
            cp_out[cur] = pltpu.async_copy(
                outb[cur], q_hbm.at[pl.ds(base + k * _CH, _CH)], semo[cur])
        cp_out[0].wait()
        cp_out[1].wait()

        pbuf[...] = acc
        pltpu.sync_copy(pbuf, part_hbm.at[pl.ds(wid * _L, _L)])

    return body(z_flat, levels)


def kernel(z, color_table):
    levels = jnp.pad(color_table[:6, 2], (0, _L - 6), mode="edge")
    q_flat, partials = _sc_quantize(z.reshape(-1), levels)
    m = jnp.sum(partials) / _N
    loss = 10.0 * m + m
    return (q_flat.reshape(z.shape), loss)


# speedup vs baseline: 1.1161x; 1.1161x over previous
import functools

import ml_dtypes
import numpy as np

import jax
import jax.numpy as jnp
from jax import lax
from jax.experimental import pallas as pl
from jax.experimental.pallas import tpu as pltpu
from jax.experimental.pallas import tpu_sc as plsc

_L = 16
_NC = 2
_NS = 16
_NW = _NC * _NS
_N = 8 * 3 * 224 * 224
_PER_W = _N // _NW
_NCH = 8
_CH = _PER_W // _NCH
_CVECS = _CH // _L


def _decision_thresholds():
    lev = np.array([0.0, 0.2, 0.4, 0.6, 0.8, 1.0], np.float64)
    t32 = lev.astype(np.float32)
    bt = t32.astype(ml_dtypes.bfloat16).astype(np.float64)
    t2 = (t32 * t32).astype(np.float32).astype(np.float64)
    B = (t2[1:] - t2[:-1]) / (2.0 * (bt[1:] - bt[:-1]))
    C = []
    for b in B:
        vb = np.float64(ml_dtypes.bfloat16(b))
        bits = np.float32(vb).view(np.uint32) >> 16
        if vb > b:
            bits -= 1
            vb = np.float64(np.array([bits << 16], np.uint32).view(np.float32)[0])
        nxt = np.float64(np.array([(bits + 1) << 16], np.uint32).view(np.float32)[0])
        mid = np.float32((vb + nxt) / 2.0)
        if bits & 1:
            C.append(float(np.nextafter(mid, np.float32(-1.0), dtype=np.float32)))
        else:
            C.append(float(mid))
    return C


_C = _decision_thresholds()


def _sc_quantize(z_flat, levels):
    mesh = plsc.VectorSubcoreMesh(core_axis_name="c", subcore_axis_name="s")

    @functools.partial(
        pl.kernel,
        mesh=mesh,
        out_type=[
            jax.ShapeDtypeStruct((_N,), jnp.float32),
            jax.ShapeDtypeStruct((_NW * _L,), jnp.float32),
        ],
        scratch_types=[
            pltpu.VMEM((_PER_W,), jnp.float32),
            pltpu.VMEM((_PER_W,), jnp.float32),
            pltpu.VMEM((_L,), jnp.float32),
            pltpu.VMEM((_L,), jnp.float32),
        ],
    )
    def body(z_hbm, lvl_hbm, q_hbm, part_hbm, xbuf, qbuf, lvlbuf, pbuf):
        wid = lax.axis_index("c") * _NS + lax.axis_index("s")
        base = wid * _PER_W
        pltpu.sync_copy(lvl_hbm, lvlbuf)
        lvl_vec = lvlbuf[...]
        pltpu.sync_copy(z_hbm.at[pl.ds(base, _PER_W)], xbuf)

        def step(j, acc):
            o = j * _L
            xv = xbuf[pl.ds(o, _L)]
            iv = (jnp.where(xv > _C[0], 1, 0)
                  + jnp.where(xv > _C[1], 1, 0)
                  + jnp.where(xv > _C[2], 1, 0)
                  + jnp.where(xv > _C[3], 1, 0)
                  + jnp.where(xv > _C[4], 1, 0))
            qv = lax.gather(
                lvl_vec, iv[:, None],
                dimension_numbers=lax.GatherDimensionNumbers(
                    offset_dims=(), collapsed_slice_dims=(0,),
                    start_index_map=(0,)),
                slice_sizes=(1,),
                mode=lax.GatherScatterMode.PROMISE_IN_BOUNDS)
            qbuf[pl.ds(o, _L)] = qv
            d = qv - xv
            return acc + d * d

        acc = lax.fori_loop(0, _PER_W // _L, step,
                            jnp.zeros((_L,), jnp.float32))
        pltpu.sync_copy(qbuf, q_hbm.at[pl.ds(base, _PER_W)])
        pbuf[...] = acc
        pltpu.sync_copy(pbuf, part_hbm.at[pl.ds(wid * _L, _L)])

    return body(z_flat, levels)


def kernel(z, color_table):
    levels = jnp.pad(color_table[:6, 2], (0, _L - 6), mode="edge")
    q_flat, partials = _sc_quantize(z.reshape(-1), levels)
    m = jnp.sum(partials) / _N
    loss = 10.0 * m + m
    return (q_flat.reshape(z.shape), loss)


# speedup vs baseline: 1.2409x; 1.1119x over previous
import functools

import ml_dtypes
import numpy as np

import jax
import jax.numpy as jnp
from jax import lax
from jax.experimental import pallas as pl
from jax.experimental.pallas import tpu as pltpu
from jax.experimental.pallas import tpu_sc as plsc

_L = 16
_NC = 2
_NS = 16
_NW = _NC * _NS
_N = 8 * 3 * 224 * 224
_PER_W = _N // _NW
_NCH = 8
_CH = _PER_W // _NCH
_CVECS = _CH // _L


def _decision_thresholds():
    lev = np.array([0.0, 0.2, 0.4, 0.6, 0.8, 1.0], np.float64)
    t32 = lev.astype(np.float32)
    bt = t32.astype(ml_dtypes.bfloat16).astype(np.float64)
    t2 = (t32 * t32).astype(np.float32).astype(np.float64)
    B = (t2[1:] - t2[:-1]) / (2.0 * (bt[1:] - bt[:-1]))
    C = []
    for b in B:
        vb = np.float64(ml_dtypes.bfloat16(b))
        bits = np.float32(vb).view(np.uint32) >> 16
        if vb > b:
            bits -= 1
            vb = np.float64(np.array([bits << 16], np.uint32).view(np.float32)[0])
        nxt = np.float64(np.array([(bits + 1) << 16], np.uint32).view(np.float32)[0])
        mid = np.float32((vb + nxt) / 2.0)
        if bits & 1:
            C.append(float(np.nextafter(mid, np.float32(-1.0), dtype=np.float32)))
        else:
            C.append(float(mid))
    return C


_C = _decision_thresholds()


def _vgather(vec, idx):
    return lax.gather(
        vec, idx[:, None],
        dimension_numbers=lax.GatherDimensionNumbers(
            offset_dims=(), collapsed_slice_dims=(0,),
            start_index_map=(0,)),
        slice_sizes=(1,),
        mode=lax.GatherScatterMode.PROMISE_IN_BOUNDS)


def _sc_quantize(z_flat, aux):
    mesh = plsc.VectorSubcoreMesh(core_axis_name="c", subcore_axis_name="s")

    @functools.partial(
        pl.kernel,
        mesh=mesh,
        out_type=[
            jax.ShapeDtypeStruct((_N,), jnp.float32),
            jax.ShapeDtypeStruct((_NW * _L,), jnp.float32),
        ],
        scratch_types=[
            pltpu.VMEM((_PER_W,), jnp.float32),
            pltpu.VMEM((_PER_W,), jnp.float32),
            pltpu.VMEM((2 * _L,), jnp.float32),
            pltpu.VMEM((_L,), jnp.float32),
        ],
    )
    def body(z_hbm, aux_hbm, q_hbm, part_hbm, xbuf, qbuf, auxbuf, pbuf):
        wid = lax.axis_index("c") * _NS + lax.axis_index("s")
        base = wid * _PER_W
        pltpu.sync_copy(aux_hbm, auxbuf)
        lvl_vec = auxbuf[pl.ds(0, _L)]
        chi_vec = auxbuf[pl.ds(_L, _L)]
        pltpu.sync_copy(z_hbm.at[pl.ds(base, _PER_W)], xbuf)

        def one(o):
            xv = xbuf[pl.ds(o, _L)]
            y = (xv * 5.0 - 0.03) + 12582912.0
            b = lax.bitcast_convert_type(y, jnp.int32)
            up = jnp.where(xv > _vgather(chi_vec, b), 1, 0)
            qv = _vgather(lvl_vec, b + up)
            qbuf[pl.ds(o, _L)] = qv
            d = qv - xv
            return d * d

        def step(j, accs):
            o = j * (2 * _L)
            a0, a1 = accs
            return a0 + one(o), a1 + one(o + _L)

        z16 = jnp.zeros((_L,), jnp.float32)
        acc0, acc1 = lax.fori_loop(0, _PER_W // (2 * _L), step, (z16, z16))
        acc = acc0 + acc1
        pltpu.sync_copy(qbuf, q_hbm.at[pl.ds(base, _PER_W)])
        pbuf[...] = acc
        pltpu.sync_copy(pbuf, part_hbm.at[pl.ds(wid * _L, _L)])

    return body(z_flat, aux)


_CHI = np.full(_L, 2.0, np.float32)
_CHI[:5] = _C


def kernel(z, color_table):
    levels = jnp.pad(color_table[:6, 2], (0, _L - 6), mode="edge")
    aux = jnp.concatenate([levels, jnp.asarray(_CHI)])
    q_flat, partials = _sc_quantize(z.reshape(-1), aux)
    m = jnp.sum(partials) / _N
    loss = 10.0 * m + m
    return (q_flat.reshape(z.shape), loss)
